# baseline (speedup 1.0000x reference)
import functools

import jax
import jax.numpy as jnp
from jax import lax
from jax.experimental import pallas as pl
from jax.experimental.pallas import tpu as pltpu
from jax.experimental.pallas import tpu_sc as plsc

_PAD = 39972

_NC = 2
_NS = 16
_NW = _NC * _NS
_L = 16

_ROW_BLK = 1024


def _proj_body(tab_ref, w_ref, b_ref, out_ref):
    i = pl.program_id(0)
    row = i * _ROW_BLK + lax.broadcasted_iota(jnp.int32, (_ROW_BLK, 1), 0)
    t = jnp.where(row != _PAD, tab_ref[...], 0.0)
    y = jnp.dot(t, w_ref[...], preferred_element_type=jnp.float32)
    out_ref[...] = jnp.maximum(y + b_ref[...], 0.0)


def _project_table(table, W, b):
    V, D = table.shape
    H = W.shape[1]
    Wp = jnp.pad(W, ((0, 0), (0, D - H)))
    bp = jnp.pad(b, (0, D - H)).reshape(1, D)
    grid = pl.cdiv(V, _ROW_BLK)
    return pl.pallas_call(
        _proj_body,
        grid=(grid,),
        in_specs=[
            pl.BlockSpec((_ROW_BLK, D), lambda i: (i, 0)),
            pl.BlockSpec((D, D), lambda i: (0, 0)),
            pl.BlockSpec((1, D), lambda i: (0, 0)),
        ],
        out_specs=pl.BlockSpec((_ROW_BLK, D), lambda i: (i, 0)),
        out_shape=jax.ShapeDtypeStruct((V, D), jnp.float32),
    )(table, Wp, bp)


def _make_gather(n, s, D, H):
    assert n % _NW == 0 and s % 2 == 0 and H % _L == 0
    bpw = n // _NW

    mesh = plsc.VectorSubcoreMesh(
        core_axis_name="c", subcore_axis_name="s",
        num_cores=_NC, num_subcores=_NS,
    )

    @functools.partial(
        pl.kernel,
        out_type=jax.ShapeDtypeStruct((s, H, n), jnp.float32),
        mesh=mesh,
        compiler_params=pltpu.CompilerParams(needs_layout_passes=False),
        scratch_types=[
            pltpu.VMEM((s, bpw), jnp.int32),
            pltpu.VMEM((bpw, D), jnp.float32),
            pltpu.VMEM((bpw, D), jnp.float32),
            pltpu.VMEM((H, bpw), jnp.float32),
            pltpu.VMEM((H, bpw), jnp.float32),
            pltpu.SemaphoreType.DMA,
            pltpu.SemaphoreType.DMA,
            pltpu.SemaphoreType.DMA,
            pltpu.SemaphoreType.DMA,
        ],
    )
    def gather(ytab_hbm, idxt_hbm, out_hbm, idx_v, ga, gb, pa, pb,
               gsa, gsb, wsa, wsb):
        wid = lax.axis_index("s") * _NC + lax.axis_index("c")
        b0 = wid * bpw
        pltpu.sync_copy(idxt_hbm.at[:, pl.ds(b0, bpw)], idx_v)

        bufg = (ga, gb)
        bufp = (pa, pb)
        gsem = (gsa, gsb)
        wsem = (wsa, wsb)

        def gather_copy(t, k):
            return pltpu.make_async_copy(
                ytab_hbm.at[idx_v.at[t]], bufg[k], gsem[k])

        def wb_copy(t, k):
            return pltpu.make_async_copy(
                bufp[k], out_hbm.at[t, :, pl.ds(b0, bpw)], wsem[k])

        gather_copy(0, 0).start()
        gather_copy(1, 1).start()

        def body(tt, carry):
            for k in (0, 1):
                t = 2 * tt + k
                gather_copy(t, k).wait()

                @pl.when(tt > 0)
                def _():
                    wb_copy(t - 2, k).wait()

                for h in range(H):
                    col = jnp.full((_L,), h, jnp.int32)
                    for g in range(bpw // _L):
                        rows = lax.iota(jnp.int32, _L) + g * _L
                        bufp[k][h, pl.ds(g * _L, _L)] = plsc.load_gather(
                            bufg[k], [rows, col])
                wb_copy(t, k).start()

                @pl.when(t + 2 < s)
                def _():
                    gather_copy(t + 2, k).start()
            return carry

        lax.fori_loop(0, s // 2, body, 0)
        wb_copy(s - 2, 0).wait()
        wb_copy(s - 1, 1).wait()

    return gather


def kernel(pcp_batch, pcp_masks, table, W, b):
    n, s = pcp_batch.shape
    H = W.shape[1]
    ytab = _project_table(table, W, b)
    idx_t = pcp_batch.T.astype(jnp.int32)
    y_t = _make_gather(n, s, table.shape[1], H)(ytab, idx_t)
    return jnp.transpose(y_t, (2, 0, 1)), pcp_masks


# speedup vs baseline: 2.7923x; 2.7923x over previous
import functools

import jax
import jax.numpy as jnp
from jax import lax
from jax.experimental import pallas as pl
from jax.experimental.pallas import tpu as pltpu
from jax.experimental.pallas import tpu_sc as plsc

_PAD = 39972

_NC = 2
_NS = 16
_NW = _NC * _NS

_CH = 128
_ROW_BLK = 1024


def _proj_body(tab_ref, w_ref, b_ref, out_ref):
    i = pl.program_id(0)
    row = i * _ROW_BLK + lax.broadcasted_iota(jnp.int32, (_ROW_BLK, 1), 0)
    t = jnp.where(row != _PAD, tab_ref[...], 0.0)
    y = jnp.dot(t, w_ref[...], preferred_element_type=jnp.float32)
    out_ref[...] = jnp.maximum(y + b_ref[...], 0.0)


def _project_table(table, W, b):
    V, D = table.shape
    H = W.shape[1]
    Wp = jnp.pad(W, ((0, 0), (0, D - H)))
    bp = jnp.pad(b, (0, D - H)).reshape(1, D)
    grid = pl.cdiv(V, _ROW_BLK)
    return pl.pallas_call(
        _proj_body,
        grid=(grid,),
        in_specs=[
            pl.BlockSpec((_ROW_BLK, D), lambda i: (i, 0)),
            pl.BlockSpec((D, D), lambda i: (0, 0)),
            pl.BlockSpec((1, D), lambda i: (0, 0)),
        ],
        out_specs=pl.BlockSpec((_ROW_BLK, D), lambda i: (i, 0)),
        out_shape=jax.ShapeDtypeStruct((V, D), jnp.float32),
    )(table, Wp, bp)


def _make_gather(B, D, H):
    assert B % (_NW * 2 * _CH) == 0
    ipw = B // _NW
    nchunk = ipw // _CH

    mesh = plsc.VectorSubcoreMesh(
        core_axis_name="c", subcore_axis_name="s",
        num_cores=_NC, num_subcores=_NS,
    )

    @functools.partial(
        pl.kernel,
        out_type=jax.ShapeDtypeStruct((B, H), jnp.float32),
        mesh=mesh,
        scratch_types=[
            pltpu.VMEM((ipw,), jnp.int32),
            pltpu.VMEM((_CH, D), jnp.float32),
            pltpu.VMEM((_CH, D), jnp.float32),
            pltpu.VMEM((_CH, H), jnp.float32),
            pltpu.VMEM((_CH, H), jnp.float32),
            pltpu.SemaphoreType.DMA,
            pltpu.SemaphoreType.DMA,
            pltpu.SemaphoreType.DMA,
            pltpu.SemaphoreType.DMA,
        ],
    )
    def gather(ytab_hbm, idx_hbm, out_hbm, idx_v, ga, gb, pa, pb,
               gsa, gsb, wsa, wsb):
        wid = lax.axis_index("s") * _NC + lax.axis_index("c")
        base = wid * ipw
        pltpu.sync_copy(idx_hbm.at[pl.ds(base, ipw)], idx_v)

        bufg = (ga, gb)
        bufp = (pa, pb)
        gsem = (gsa, gsb)
        wsem = (wsa, wsb)

        def gather_copy(j, k):
            return pltpu.make_async_copy(
                ytab_hbm.at[idx_v.at[pl.ds(j * _CH, _CH)]], bufg[k], gsem[k])

        def wb_copy(j, k):
            return pltpu.make_async_copy(
                bufp[k], out_hbm.at[pl.ds(base + j * _CH, _CH)], wsem[k])

        gather_copy(0, 0).start()
        gather_copy(1, 1).start()

        def body(jj, carry):
            for k in (0, 1):
                j = 2 * jj + k
                gather_copy(j, k).wait()

                @pl.when(jj > 0)
                def _():
                    wb_copy(j - 2, k).wait()

                def compact(r8, c2):
                    for q in range(8):
                        for c in range(H // 16):
                            sl = pl.ds(c * 16, 16)
                            bufp[k][r8 * 8 + q, sl] = bufg[k][r8 * 8 + q, sl]
                    return c2

                lax.fori_loop(0, _CH // 8, compact, 0)
                wb_copy(j, k).start()

                @pl.when(j + 2 < nchunk)
                def _():
                    gather_copy(j + 2, k).start()
            return carry

        lax.fori_loop(0, nchunk // 2, body, 0)
        wb_copy(nchunk - 2, 0).wait()
        wb_copy(nchunk - 1, 1).wait()

    return gather


def kernel(pcp_batch, pcp_masks, table, W, b):
    n, s = pcp_batch.shape
    H = W.shape[1]
    ytab = _project_table(table, W, b)
    idx = pcp_batch.reshape(-1).astype(jnp.int32)
    y = _make_gather(n * s, table.shape[1], H)(ytab, idx)
    return y.reshape(n, s, H), pcp_masks
